# Initial kernel scaffold; baseline (speedup 1.0000x reference)
import jax
import jax.numpy as jnp
from jax.experimental import pallas as pl


def kernel(input_sequences, table):
    raise NotImplementedError("write your pallas kernel here")


# speedup vs baseline: 8.5245x; 8.5245x over previous
import functools

import jax
import jax.numpy as jnp
from jax import lax
from jax.experimental import pallas as pl
from jax.experimental.pallas import tpu as pltpu
from jax.experimental.pallas import tpu_sc as plsc

_info = plsc.get_sparse_core_info()
NC, NS, L = _info.num_cores, _info.num_subcores, _info.num_lanes
NW = NC * NS

CH = 128
K = 4


@functools.partial(jax.jit, static_argnames=("G",))
def _embedding_gather(idx, table, G):
    V, D = table.shape
    N = NW * G * CH
    nblk = G // K
    mesh = plsc.VectorSubcoreMesh(core_axis_name="c", subcore_axis_name="s")

    @functools.partial(
        pl.kernel,
        out_type=jax.ShapeDtypeStruct((N, D), jnp.float32),
        mesh=mesh,
        scratch_types=[
            pltpu.VMEM((G, CH), jnp.int32),
            pltpu.VMEM((K, CH, D), jnp.float32),
            pltpu.SemaphoreType.DMA,
        ],
    )
    def k(idx_hbm, table_hbm, out_hbm, idx_v, rows_v, gsem):
        wid = lax.axis_index("s") * NC + lax.axis_index("c")
        base = wid * (G * CH)
        pltpu.sync_copy(idx_hbm.at[wid], idx_v)

        @pl.loop(0, nblk)
        def _(blk):
            g0 = blk * K
            cps = []
            for j in range(K):
                cps.append(
                    pltpu.async_copy(
                        table_hbm.at[idx_v.at[g0 + j]], rows_v.at[j], gsem
                    )
                )
            for j in range(K):
                cps[j].wait()
            for j in range(K):
                pltpu.sync_copy(
                    rows_v.at[j],
                    out_hbm.at[pl.ds(base + (g0 + j) * CH, CH)],
                )

    return k(idx, table)


def kernel(input_sequences, table):
    B, S = input_sequences.shape
    V, D = table.shape
    N = B * S
    assert N % (NW * CH) == 0
    G = N // (NW * CH)
    idx = input_sequences.reshape(NW, G, CH).astype(jnp.int32)
    out = _embedding_gather(idx, table, G)
    return out.reshape(B, S, D)


# speedup vs baseline: 9.2123x; 1.0807x over previous
import functools

import jax
import jax.numpy as jnp
from jax import lax
from jax.experimental import pallas as pl
from jax.experimental.pallas import tpu as pltpu
from jax.experimental.pallas import tpu_sc as plsc

_info = plsc.get_sparse_core_info()
NC, NS, L = _info.num_cores, _info.num_subcores, _info.num_lanes
NW = NC * NS

CH = 128
K = 2


@functools.partial(jax.jit, static_argnames=("G",))
def _embedding_gather(idx, table, G):
    V, D = table.shape
    N = NW * G * CH
    nblk = G // K
    assert nblk % 2 == 0 and nblk >= 4
    mesh = plsc.VectorSubcoreMesh(core_axis_name="c", subcore_axis_name="s")

    @functools.partial(
        pl.kernel,
        out_type=jax.ShapeDtypeStruct((N, D), jnp.float32),
        mesh=mesh,
        scratch_types=[
            pltpu.VMEM((G, CH), jnp.int32),
            pltpu.VMEM((K * CH, D), jnp.float32),
            pltpu.VMEM((K * CH, D), jnp.float32),
            pltpu.SemaphoreType.DMA,
            pltpu.SemaphoreType.DMA,
            pltpu.SemaphoreType.DMA,
            pltpu.SemaphoreType.DMA,
        ],
    )
    def k(idx_hbm, table_hbm, out_hbm, idx_v, rows0, rows1, gs0, gs1, os0, os1):
        wid = lax.axis_index("s") * NC + lax.axis_index("c")
        base = wid * (G * CH)
        rows = (rows0, rows1)
        gsem = (gs0, gs1)
        osem = (os0, os1)
        pltpu.sync_copy(idx_hbm.at[wid], idx_v)

        def issue_gathers(blk, p):
            for j in range(K):
                pltpu.async_copy(
                    table_hbm.at[idx_v.at[blk * K + j]],
                    rows[p].at[pl.ds(j * CH, CH)],
                    gsem[p],
                )

        def drain_gathers(p):
            pltpu.make_async_copy(
                table_hbm.at[pl.ds(0, K * CH)], rows[p], gsem[p]
            ).wait()

        def out_slice(blk):
            return out_hbm.at[pl.ds(base + blk * (K * CH), K * CH)]

        def issue_out(blk, p):
            pltpu.async_copy(rows[p], out_slice(blk), osem[p])

        def drain_out(blk, p):
            pltpu.make_async_copy(rows[p], out_slice(blk), osem[p]).wait()

        issue_gathers(0, 0)
        issue_gathers(1, 1)
        drain_gathers(0)
        issue_out(0, 0)

        @pl.loop(0, (nblk - 2) // 2)
        def _(pair):
            for q in range(2):
                blk = 2 + pair * 2 + q
                p = q
                drain_out(blk - 2, p)
                issue_gathers(blk, p)
                drain_gathers(p ^ 1)
                issue_out(blk - 1, p ^ 1)

        p_last = (nblk - 1) % 2
        drain_gathers(p_last)
        issue_out(nblk - 1, p_last)
        drain_out(nblk - 2, p_last ^ 1)
        drain_out(nblk - 1, p_last)

    return k(idx, table)


def kernel(input_sequences, table):
    B, S = input_sequences.shape
    V, D = table.shape
    N = B * S
    assert N % (NW * CH) == 0
    G = N // (NW * CH)
    idx = input_sequences.reshape(NW, G, CH).astype(jnp.int32)
    out = _embedding_gather(idx, table, G)
    return out.reshape(B, S, D)


# speedup vs baseline: 9.2758x; 1.0069x over previous
import functools

import jax
import jax.numpy as jnp
from jax import lax
from jax.experimental import pallas as pl
from jax.experimental.pallas import tpu as pltpu
from jax.experimental.pallas import tpu_sc as plsc

_info = plsc.get_sparse_core_info()
NC, NS, L = _info.num_cores, _info.num_subcores, _info.num_lanes
NW = NC * NS

CH = 64
K = 2
NBUF = 4


@functools.partial(jax.jit, static_argnames=("G",))
def _embedding_gather(idx, table, G):
    V, D = table.shape
    N = NW * G * CH
    nblk = G // K
    assert nblk % NBUF == 0 and nblk >= 2 * NBUF
    mesh = plsc.VectorSubcoreMesh(core_axis_name="c", subcore_axis_name="s")

    @functools.partial(
        pl.kernel,
        out_type=jax.ShapeDtypeStruct((N, D), jnp.float32),
        mesh=mesh,
        scratch_types=[
            pltpu.VMEM((G * CH,), jnp.int32),
            [pltpu.VMEM((K * CH, D), jnp.float32) for _ in range(NBUF)],
            [pltpu.SemaphoreType.DMA for _ in range(NBUF)],
            [pltpu.SemaphoreType.DMA for _ in range(NBUF)],
        ],
    )
    def k(idx_hbm, table_hbm, out_hbm, idx_v, rows, gsem, osem):
        wid = lax.axis_index("s") * NC + lax.axis_index("c")
        base = wid * (G * CH)
        pltpu.sync_copy(idx_hbm.at[wid], idx_v)

        def issue_gathers(blk, p):
            for j in range(K):
                pltpu.async_copy(
                    table_hbm.at[idx_v.at[pl.ds((blk * K + j) * CH, CH)]],
                    rows[p].at[pl.ds(j * CH, CH)],
                    gsem[p],
                )

        def drain_gathers(p):
            pltpu.make_async_copy(
                table_hbm.at[pl.ds(0, K * CH)], rows[p], gsem[p]
            ).wait()

        def out_slice(blk):
            return out_hbm.at[pl.ds(base + blk * (K * CH), K * CH)]

        def issue_out(blk, p):
            pltpu.async_copy(rows[p], out_slice(blk), osem[p])

        def drain_out(blk, p):
            pltpu.make_async_copy(rows[p], out_slice(blk), osem[p]).wait()

        issue_gathers(0, 0)
        for blk in range(1, NBUF):
            issue_gathers(blk, blk)
            drain_gathers(blk - 1)
            issue_out(blk - 1, blk - 1)

        @pl.loop(0, (nblk - NBUF) // NBUF)
        def _(grp):
            for q in range(NBUF):
                blk = NBUF + grp * NBUF + q
                drain_out(blk - NBUF, q)
                issue_gathers(blk, q)
                drain_gathers((q - 1) % NBUF)
                issue_out(blk - 1, (q - 1) % NBUF)

        p_last = (nblk - 1) % NBUF
        drain_gathers(p_last)
        issue_out(nblk - 1, p_last)
        for blk in range(nblk - NBUF, nblk):
            drain_out(blk, blk % NBUF)

    return k(idx, table)


def kernel(input_sequences, table):
    B, S = input_sequences.shape
    V, D = table.shape
    N = B * S
    assert N % (NW * CH) == 0
    G = N // (NW * CH)
    idx = input_sequences.reshape(NW, G * CH).astype(jnp.int32)
    out = _embedding_gather(idx, table, G)
    return out.reshape(B, S, D)


# speedup vs baseline: 9.3476x; 1.0077x over previous
import functools

import jax
import jax.numpy as jnp
from jax import lax
from jax.experimental import pallas as pl
from jax.experimental.pallas import tpu as pltpu
from jax.experimental.pallas import tpu_sc as plsc

_info = plsc.get_sparse_core_info()
NC, NS, L = _info.num_cores, _info.num_subcores, _info.num_lanes
NW = NC * NS

CH = 80
K = 2
NBUF = 5


@functools.partial(jax.jit, static_argnames=("G",))
def _embedding_gather(idx, table, G):
    V, D = table.shape
    N = NW * G * CH
    nblk = G // K
    assert nblk % NBUF == 0 and nblk >= 2 * NBUF
    mesh = plsc.VectorSubcoreMesh(core_axis_name="c", subcore_axis_name="s")

    @functools.partial(
        pl.kernel,
        out_type=jax.ShapeDtypeStruct((N, D), jnp.float32),
        mesh=mesh,
        scratch_types=[
            pltpu.VMEM((G * CH,), jnp.int32),
            [pltpu.VMEM((K * CH, D), jnp.float32) for _ in range(NBUF)],
            [pltpu.SemaphoreType.DMA for _ in range(NBUF)],
            [pltpu.SemaphoreType.DMA for _ in range(NBUF)],
        ],
    )
    def k(idx_hbm, table_hbm, out_hbm, idx_v, rows, gsem, osem):
        wid = lax.axis_index("s") * NC + lax.axis_index("c")
        base = wid * (G * CH)
        pltpu.sync_copy(idx_hbm.at[wid], idx_v)

        def issue_gathers(blk, p):
            for j in range(K):
                pltpu.async_copy(
                    table_hbm.at[idx_v.at[pl.ds((blk * K + j) * CH, CH)]],
                    rows[p].at[pl.ds(j * CH, CH)],
                    gsem[p],
                )

        def drain_gathers(p):
            pltpu.make_async_copy(
                table_hbm.at[pl.ds(0, K * CH)], rows[p], gsem[p]
            ).wait()

        def out_slice(blk):
            return out_hbm.at[pl.ds(base + blk * (K * CH), K * CH)]

        def issue_out(blk, p):
            pltpu.async_copy(rows[p], out_slice(blk), osem[p])

        def drain_out(blk, p):
            pltpu.make_async_copy(rows[p], out_slice(blk), osem[p]).wait()

        issue_gathers(0, 0)
        for blk in range(1, NBUF):
            issue_gathers(blk, blk)
            drain_gathers(blk - 1)
            issue_out(blk - 1, blk - 1)

        @pl.loop(0, (nblk - NBUF) // NBUF)
        def _(grp):
            for q in range(NBUF):
                blk = NBUF + grp * NBUF + q
                drain_out(blk - NBUF, q)
                issue_gathers(blk, q)
                drain_gathers((q - 1) % NBUF)
                issue_out(blk - 1, (q - 1) % NBUF)

        p_last = (nblk - 1) % NBUF
        drain_gathers(p_last)
        issue_out(nblk - 1, p_last)
        for blk in range(nblk - NBUF, nblk):
            drain_out(blk, blk % NBUF)

    return k(idx, table)


def kernel(input_sequences, table):
    B, S = input_sequences.shape
    V, D = table.shape
    N = B * S
    assert N % (NW * CH) == 0
    G = N // (NW * CH)
    idx = input_sequences.reshape(NW, G * CH).astype(jnp.int32)
    out = _embedding_gather(idx, table, G)
    return out.reshape(B, S, D)
